# Initial kernel scaffold; baseline (speedup 1.0000x reference)
import jax
import jax.numpy as jnp
from jax.experimental import pallas as pl


def kernel(kv_cache, num_evicts, cachelens, num_sink_tokens, n_local_heads, head_dim):
    raise NotImplementedError("write your pallas kernel here")


# speedup vs baseline: 7.9103x; 7.9103x over previous
import functools

import jax
import jax.numpy as jnp
from jax import lax
from jax.experimental import pallas as pl
from jax.experimental.pallas import tpu as pltpu
from jax.experimental.pallas import tpu_sc as plsc

_NC, _NS, _L = 2, 16, 16
_NW = _NC * _NS
_CHUNK = 32


def _sc_shift_gather(rows_hbm, params_hbm, *, n_rows, d):
    rows_per_w = n_rows // _NW
    n_chunks = rows_per_w // _CHUNK
    mesh = plsc.VectorSubcoreMesh(core_axis_name="c", subcore_axis_name="s")

    @functools.partial(
        pl.kernel,
        mesh=mesh,
        out_type=jax.ShapeDtypeStruct((n_rows, d), jnp.float32),
        scratch_types=[
            pltpu.VMEM((3, _L), jnp.int32),
            pltpu.VMEM((_CHUNK,), jnp.int32),
            pltpu.VMEM((_CHUNK,), jnp.int32),
            pltpu.VMEM((_CHUNK, d), jnp.float32),
            pltpu.VMEM((_CHUNK, d), jnp.float32),
            pltpu.SemaphoreType.DMA,
            pltpu.SemaphoreType.DMA,
            pltpu.SemaphoreType.DMA,
            pltpu.SemaphoreType.DMA,
        ],
    )
    def k(rows, params_h, out,
          params_v, idx0, idx1, buf0, buf1, gs0, gs1, ss0, ss1):
        wid = lax.axis_index("s") * _NC + lax.axis_index("c")
        b = wid // 4
        row_base = wid * rows_per_w

        pltpu.sync_copy(params_h.at[wid], params_v)
        evict = params_v[0, :]
        target = params_v[1, :]
        sink = params_v[2, :]
        lane = lax.iota(jnp.int32, _L)

        idx_slots = (idx0, idx1)
        buf_slots = (buf0, buf1)
        gsem = (gs0, gs1)
        ssem = (ss0, ss1)

        def fill_idx(j, slot):
            for g in range(_CHUNK // _L):
                r = row_base + j * _CHUNK + g * _L + lane
                tok = r & 15
                plane = (r >> 4) & 1
                t = ((r >> 5) & 127) * 16 + tok
                in_region = (t >= sink) & (t < target)
                src_t = jnp.where(in_region, t + evict, t)
                src_r = ((b * 4096)
                         + ((src_t >> 4) << 5)
                         + (plane << 4)
                         + (src_t & 15))
                idx_slots[slot][pl.ds(g * _L, _L)] = src_r

        def g_start(j, slot):
            return pltpu.async_copy(
                rows.at[idx_slots[slot]], buf_slots[slot], gsem[slot])

        def s_start(j, slot):
            return pltpu.async_copy(
                buf_slots[slot],
                out.at[pl.ds(row_base + j * _CHUNK, _CHUNK)],
                ssem[slot])

        fill_idx(0, 0)
        g_prev = g_start(0, 0)
        s_pend = [None, None]
        for j in range(n_chunks):
            slot = j & 1
            nslot = slot ^ 1
            if j + 1 < n_chunks:
                if s_pend[nslot] is not None:
                    s_pend[nslot].wait()
                    s_pend[nslot] = None
                fill_idx(j + 1, nslot)
                g_next = g_start(j + 1, nslot)
            g_prev.wait()
            s_pend[slot] = s_start(j, slot)
            if j + 1 < n_chunks:
                g_prev = g_next
        for p in s_pend:
            if p is not None:
                p.wait()

    return k(rows_hbm, params_hbm)


def kernel(kv_cache, num_evicts, cachelens, num_sink_tokens, n_local_heads, head_dim):
    P, two, S, H, Dh = kv_cache.shape
    bsz = num_evicts.shape[0]
    n_rows = P * two * S
    d = H * Dh

    rows = kv_cache.reshape(n_rows, d)
    evict = num_evicts.astype(jnp.int32)
    target = cachelens.astype(jnp.int32) - evict
    wb = jnp.arange(_NW, dtype=jnp.int32) // (_NW // bsz)
    params = jnp.stack(
        [evict[wb], target[wb],
         jnp.full((_NW,), num_sink_tokens, jnp.int32)], axis=1)
    params = jnp.broadcast_to(params[:, :, None], (_NW, 3, _L)).astype(jnp.int32)

    out = _sc_shift_gather(rows, params, n_rows=n_rows, d=d)
    return out.reshape(P, two, S, H, Dh)


# speedup vs baseline: 21.3751x; 2.7022x over previous
import functools

import jax
import jax.numpy as jnp
from jax import lax
from jax.experimental import pallas as pl
from jax.experimental.pallas import tpu as pltpu
from jax.experimental.pallas import tpu_sc as plsc

_NC, _NS, _L = 2, 16, 16
_NW = _NC * _NS
_CHUNK = 32


def _sc_shift_gather(rows_hbm, params_hbm, *, n_rows, d):
    rows_per_w = n_rows // _NW
    n_chunks = rows_per_w // _CHUNK
    mesh = plsc.VectorSubcoreMesh(core_axis_name="c", subcore_axis_name="s")

    @functools.partial(
        pl.kernel,
        mesh=mesh,
        out_type=jax.ShapeDtypeStruct((n_rows, 8, d // 8), jnp.float32),
        scratch_types=[
            pltpu.VMEM((3, _L), jnp.int32),
            pltpu.VMEM((_CHUNK,), jnp.int32),
            pltpu.VMEM((_CHUNK,), jnp.int32),
            pltpu.VMEM((_CHUNK, 8, d // 8), jnp.float32),
            pltpu.VMEM((_CHUNK, 8, d // 8), jnp.float32),
            pltpu.SemaphoreType.DMA,
            pltpu.SemaphoreType.DMA,
            pltpu.SemaphoreType.DMA,
            pltpu.SemaphoreType.DMA,
        ],
    )
    def k(rows, params_h, out,
          params_v, idx0, idx1, buf0, buf1, gs0, gs1, ss0, ss1):
        wid = lax.axis_index("s") * _NC + lax.axis_index("c")
        b = wid // 4
        row_base = wid * rows_per_w

        pltpu.sync_copy(params_h.at[wid], params_v)
        evict = params_v[0, :]
        target = params_v[1, :]
        sink = params_v[2, :]
        lane = lax.iota(jnp.int32, _L)

        idx_slots = (idx0, idx1)
        buf_slots = (buf0, buf1)
        gsem = (gs0, gs1)
        ssem = (ss0, ss1)

        def fill_idx(j, slot):
            for g in range(_CHUNK // _L):
                r = row_base + j * _CHUNK + g * _L + lane
                tok = r & 15
                plane = (r >> 4) & 1
                t = ((r >> 5) & 127) * 16 + tok
                in_region = (t >= sink) & (t < target)
                src_t = jnp.where(in_region, t + evict, t)
                src_r = ((b * 4096)
                         + ((src_t >> 4) << 5)
                         + (plane << 4)
                         + (src_t & 15))
                idx_slots[slot][pl.ds(g * _L, _L)] = src_r

        def g_start(j, slot):
            return pltpu.async_copy(
                rows.at[idx_slots[slot]], buf_slots[slot], gsem[slot])

        def s_start(j, slot):
            return pltpu.async_copy(
                buf_slots[slot],
                out.at[pl.ds(row_base + j * _CHUNK, _CHUNK)],
                ssem[slot])

        fill_idx(0, 0)
        g_prev = g_start(0, 0)
        s_pend = [None, None]
        for j in range(n_chunks):
            slot = j & 1
            nslot = slot ^ 1
            if j + 1 < n_chunks:
                if s_pend[nslot] is not None:
                    s_pend[nslot].wait()
                    s_pend[nslot] = None
                fill_idx(j + 1, nslot)
                g_next = g_start(j + 1, nslot)
            g_prev.wait()
            s_pend[slot] = s_start(j, slot)
            if j + 1 < n_chunks:
                g_prev = g_next
        for p in s_pend:
            if p is not None:
                p.wait()

    return k(rows_hbm, params_hbm)


def kernel(kv_cache, num_evicts, cachelens, num_sink_tokens, n_local_heads, head_dim):
    P, two, S, H, Dh = kv_cache.shape
    bsz = num_evicts.shape[0]
    n_rows = P * two * S
    d = H * Dh

    rows = kv_cache.reshape(n_rows, H, Dh)
    evict = num_evicts.astype(jnp.int32)
    target = cachelens.astype(jnp.int32) - evict
    wb = jnp.arange(_NW, dtype=jnp.int32) // (_NW // bsz)
    params = jnp.stack(
        [evict[wb], target[wb],
         jnp.full((_NW,), num_sink_tokens, jnp.int32)], axis=1)
    params = jnp.broadcast_to(params[:, :, None], (_NW, 3, _L)).astype(jnp.int32)

    out = _sc_shift_gather(rows, params, n_rows=n_rows, d=d)
    return out.reshape(P, two, S, H, Dh)


# speedup vs baseline: 21.4343x; 1.0028x over previous
import functools

import jax
import jax.numpy as jnp
from jax import lax
from jax.experimental import pallas as pl
from jax.experimental.pallas import tpu as pltpu
from jax.experimental.pallas import tpu_sc as plsc

_NC, _NS, _L = 2, 16, 16
_NW = _NC * _NS
_CHUNK = 32
_NBUF = 3


def _sc_shift_gather(rows_hbm, params_hbm, *, n_rows, d):
    rows_per_w = n_rows // _NW
    n_chunks = rows_per_w // _CHUNK
    mesh = plsc.VectorSubcoreMesh(core_axis_name="c", subcore_axis_name="s")

    @functools.partial(
        pl.kernel,
        mesh=mesh,
        out_type=jax.ShapeDtypeStruct((n_rows, 8, d // 8), jnp.float32),
        scratch_types=(
            [pltpu.VMEM((3, _L), jnp.int32)]
            + [pltpu.VMEM((_CHUNK,), jnp.int32) for _ in range(_NBUF)]
            + [pltpu.VMEM((_CHUNK, 8, d // 8), jnp.float32) for _ in range(_NBUF)]
            + [pltpu.SemaphoreType.DMA for _ in range(2 * _NBUF)]
        ),
    )
    def k(rows, params_h, out, params_v, *ring):
        idx_slots = ring[:_NBUF]
        buf_slots = ring[_NBUF:2 * _NBUF]
        gsem = ring[2 * _NBUF:3 * _NBUF]
        ssem = ring[3 * _NBUF:]
        wid = lax.axis_index("s") * _NC + lax.axis_index("c")
        b = wid // 4
        row_base = wid * rows_per_w

        pltpu.sync_copy(params_h.at[wid], params_v)
        evict = params_v[0, :]
        target = params_v[1, :]
        sink = params_v[2, :]
        lane = lax.iota(jnp.int32, _L)

        def fill_idx(j, slot):
            for g in range(_CHUNK // _L):
                r = row_base + j * _CHUNK + g * _L + lane
                tok = r & 15
                plane = (r >> 4) & 1
                t = ((r >> 5) & 127) * 16 + tok
                in_region = (t >= sink) & (t < target)
                src_t = jnp.where(in_region, t + evict, t)
                src_r = ((b * 4096)
                         + ((src_t >> 4) << 5)
                         + (plane << 4)
                         + (src_t & 15))
                idx_slots[slot][pl.ds(g * _L, _L)] = src_r

        def g_start(j, slot):
            return pltpu.async_copy(
                rows.at[idx_slots[slot]], buf_slots[slot], gsem[slot])

        def s_start(j, slot):
            return pltpu.async_copy(
                buf_slots[slot],
                out.at[pl.ds(row_base + j * _CHUNK, _CHUNK)],
                ssem[slot])

        g_pend = [None] * _NBUF
        s_pend = [None] * _NBUF
        for j in range(min(_NBUF, n_chunks)):
            fill_idx(j, j)
            g_pend[j] = g_start(j, j)
        for j in range(n_chunks):
            slot = j % _NBUF
            g_pend[slot].wait()
            s_pend[slot] = s_start(j, slot)
            jn = j + _NBUF
            if jn < n_chunks:
                s_pend[slot].wait()
                s_pend[slot] = None
                fill_idx(jn, slot)
                g_pend[slot] = g_start(jn, slot)
        for p in s_pend:
            if p is not None:
                p.wait()

    return k(rows_hbm, params_hbm)


def kernel(kv_cache, num_evicts, cachelens, num_sink_tokens, n_local_heads, head_dim):
    P, two, S, H, Dh = kv_cache.shape
    bsz = num_evicts.shape[0]
    n_rows = P * two * S
    d = H * Dh

    rows = kv_cache.reshape(n_rows, H, Dh)
    evict = num_evicts.astype(jnp.int32)
    target = cachelens.astype(jnp.int32) - evict
    wb = jnp.arange(_NW, dtype=jnp.int32) // (_NW // bsz)
    params = jnp.stack(
        [evict[wb], target[wb],
         jnp.full((_NW,), num_sink_tokens, jnp.int32)], axis=1)
    params = jnp.broadcast_to(params[:, :, None], (_NW, 3, _L)).astype(jnp.int32)

    out = _sc_shift_gather(rows, params, n_rows=n_rows, d=d)
    return out.reshape(P, two, S, H, Dh)
